# Initial kernel scaffold; baseline (speedup 1.0000x reference)
import jax
import jax.numpy as jnp
from jax.experimental import pallas as pl


def kernel(x, pos_table):
    raise NotImplementedError("write your pallas kernel here")


# speedup vs baseline: 1.4450x; 1.4450x over previous
import functools
import math

import jax
import jax.numpy as jnp
from jax.experimental import pallas as pl


def _pe_block(x_ref, pos_ref, o_ref, *, scale):
    o_ref[...] = x_ref[...] * scale + pos_ref[...][None, :, :]


@functools.partial(jax.jit, static_argnames=("block_s",))
def _pe(x, pos_table, block_s=512):
    batch, seq_len, d_model = x.shape
    scale = math.sqrt(float(d_model))
    grid = (seq_len // block_s, batch)
    return pl.pallas_call(
        functools.partial(_pe_block, scale=scale),
        grid=grid,
        in_specs=[
            pl.BlockSpec((1, block_s, d_model), lambda s, b: (b, s, 0)),
            pl.BlockSpec((block_s, d_model), lambda s, b: (s, 0)),
        ],
        out_specs=pl.BlockSpec((1, block_s, d_model), lambda s, b: (b, s, 0)),
        out_shape=jax.ShapeDtypeStruct(x.shape, x.dtype),
    )(x, pos_table)


def kernel(x, pos_table):
    return _pe(x, pos_table)


# speedup vs baseline: 1.8012x; 1.2465x over previous
import functools
import math

import jax
import jax.numpy as jnp
from jax.experimental import pallas as pl


def _pe_block(x_ref, pos_ref, o_ref, *, scale):
    o_ref[...] = x_ref[...] * scale + pos_ref[...][None, :, :]


@functools.partial(jax.jit, static_argnames=("block_s",))
def _pe(x, pos_table, block_s=512):
    batch, seq_len, d_model = x.shape
    scale = math.sqrt(float(d_model))
    grid = (seq_len // block_s,)
    return pl.pallas_call(
        functools.partial(_pe_block, scale=scale),
        grid=grid,
        in_specs=[
            pl.BlockSpec((batch, block_s, d_model), lambda s: (0, s, 0)),
            pl.BlockSpec((block_s, d_model), lambda s: (s, 0)),
        ],
        out_specs=pl.BlockSpec((batch, block_s, d_model), lambda s: (0, s, 0)),
        out_shape=jax.ShapeDtypeStruct(x.shape, x.dtype),
    )(x, pos_table)


def kernel(x, pos_table):
    return _pe(x, pos_table)
